# baseline (speedup 1.0000x reference)
import functools

import jax
import jax.numpy as jnp
from jax import lax
from jax.experimental import pallas as pl
from jax.experimental.pallas import tpu as pltpu
from jax.experimental.pallas import tpu_sc as plsc

B = 16384
D = 32
NCAT = 8
DDENSE = 13
DOUT = NCAT * D + DDENSE

NC = 2
NS = 16
NW = NC * NS
BPW = B // NW

_mesh = plsc.VectorSubcoreMesh(
    core_axis_name="c", subcore_axis_name="s", num_cores=NC, num_subcores=NS
)


@functools.partial(
    pl.kernel,
    out_type=jax.ShapeDtypeStruct((B, DOUT), jnp.float32),
    mesh=_mesh,
    scratch_types=[
        pltpu.VMEM((BPW,), jnp.int32),
        pltpu.VMEM((BPW, D), jnp.float32),
        pltpu.VMEM((BPW, DDENSE), jnp.float32),
        pltpu.SemaphoreType.DMA,
    ],
    compiler_params=pltpu.CompilerParams(use_tc_tiling_on_sc=False),
)
def _lookup_concat(cat0, cat1, cat2, cat3, cat4, cat5, cat6, cat7, dense,
                   emb0, emb1, emb2, emb3, emb4, emb5, emb6, emb7,
                   out_hbm, idx_v, rows_v, dense_v, sem):
    cats = [cat0, cat1, cat2, cat3, cat4, cat5, cat6, cat7]
    embs = [emb0, emb1, emb2, emb3, emb4, emb5, emb6, emb7]
    wid = lax.axis_index("s") * NC + lax.axis_index("c")
    base = wid * BPW

    pltpu.sync_copy(dense.at[pl.ds(base, BPW)], dense_v)
    pltpu.sync_copy(dense_v, out_hbm.at[pl.ds(base, BPW), pl.ds(NCAT * D, DDENSE)])

    for t in range(NCAT):
        pltpu.sync_copy(cats[t].at[pl.ds(base, BPW)], idx_v)
        pltpu.async_copy(embs[t].at[idx_v], rows_v, sem).wait()
        pltpu.sync_copy(rows_v, out_hbm.at[pl.ds(base, BPW), pl.ds(t * D, D)])


def kernel(cat0, cat1, cat2, cat3, cat4, cat5, cat6, cat7, dense,
           emb0, emb1, emb2, emb3, emb4, emb5, emb6, emb7):
    return _lookup_concat(cat0, cat1, cat2, cat3, cat4, cat5, cat6, cat7, dense,
                          emb0, emb1, emb2, emb3, emb4, emb5, emb6, emb7)


# speedup vs baseline: 1.0139x; 1.0139x over previous
import functools

import jax
import jax.numpy as jnp
from jax import lax
from jax.experimental import pallas as pl
from jax.experimental.pallas import tpu as pltpu
from jax.experimental.pallas import tpu_sc as plsc

B = 16384
D = 32
NCAT = 8
DDENSE = 13
DOUT = NCAT * D + DDENSE

NC = 2
NS = 16
NW = NC * NS
BPW = B // NW
CH = 128
NCHUNK = BPW // CH
NSLOT = 2

_mesh = plsc.VectorSubcoreMesh(
    core_axis_name="c", subcore_axis_name="s", num_cores=NC, num_subcores=NS
)


@functools.partial(
    pl.kernel,
    out_type=jax.ShapeDtypeStruct((B, DOUT), jnp.float32),
    mesh=_mesh,
    scratch_types=[
        pltpu.VMEM((NCAT, BPW), jnp.int32),
        pltpu.VMEM((NSLOT, NCAT, CH, D), jnp.float32),
        pltpu.VMEM((NSLOT, CH, DDENSE), jnp.float32),
        pltpu.SemaphoreType.DMA,
        pltpu.SemaphoreType.DMA,
        pltpu.SemaphoreType.DMA,
    ],
    compiler_params=pltpu.CompilerParams(use_tc_tiling_on_sc=False),
)
def _lookup_concat(cat0, cat1, cat2, cat3, cat4, cat5, cat6, cat7, dense,
                   emb0, emb1, emb2, emb3, emb4, emb5, emb6, emb7,
                   out_hbm, idx_v, rows_v, dense_v, sem_i, sem_g, sem_w):
    cats = [cat0, cat1, cat2, cat3, cat4, cat5, cat6, cat7]
    embs = [emb0, emb1, emb2, emb3, emb4, emb5, emb6, emb7]
    wid = lax.axis_index("s") * NC + lax.axis_index("c")
    base = wid * BPW

    idx_cps = [
        pltpu.async_copy(cats[t].at[pl.ds(base, BPW)], idx_v.at[t], sem_i)
        for t in range(NCAT)
    ]
    for cp in idx_cps:
        cp.wait()

    pending_writes = [None] * NSLOT
    for k in range(NCHUNK):
        slot = k % NSLOT
        if pending_writes[slot] is not None:
            for cp in pending_writes[slot]:
                cp.wait()
        row0 = base + k * CH
        cps = [
            pltpu.async_copy(
                embs[t].at[idx_v.at[t, pl.ds(k * CH, CH)]],
                rows_v.at[slot, t],
                sem_g,
            )
            for t in range(NCAT)
        ]
        cps.append(
            pltpu.async_copy(dense.at[pl.ds(row0, CH)], dense_v.at[slot], sem_g)
        )
        for cp in cps:
            cp.wait()
        writes = [
            pltpu.async_copy(
                rows_v.at[slot, t],
                out_hbm.at[pl.ds(row0, CH), pl.ds(t * D, D)],
                sem_w,
            )
            for t in range(NCAT)
        ]
        writes.append(
            pltpu.async_copy(
                dense_v.at[slot],
                out_hbm.at[pl.ds(row0, CH), pl.ds(NCAT * D, DDENSE)],
                sem_w,
            )
        )
        pending_writes[slot] = writes
    for writes in pending_writes:
        if writes is not None:
            for cp in writes:
                cp.wait()


def kernel(cat0, cat1, cat2, cat3, cat4, cat5, cat6, cat7, dense,
           emb0, emb1, emb2, emb3, emb4, emb5, emb6, emb7):
    return _lookup_concat(cat0, cat1, cat2, cat3, cat4, cat5, cat6, cat7, dense,
                          emb0, emb1, emb2, emb3, emb4, emb5, emb6, emb7)
